# baseline (speedup 1.0000x reference)
import functools

import jax
import jax.numpy as jnp
from jax import lax
from jax.experimental import pallas as pl
from jax.experimental.pallas import tpu as pltpu
from jax.experimental.pallas import tpu_sc as plsc

N_EMBD = 64
POS_ROWS = 32
NW = 32
NBUF = 2
TH = 16
LANES = 16


def _make_gather(B, T, C):
    b_per_w = B // NW
    nb = b_per_w * (T // TH)
    mesh = plsc.VectorSubcoreMesh(core_axis_name="c", subcore_axis_name="s")

    @functools.partial(
        pl.kernel,
        mesh=mesh,
        compiler_params=pltpu.CompilerParams(use_tc_tiling_on_sc=False),
        out_type=jax.ShapeDtypeStruct((B, T, C, N_EMBD), jnp.float32),
        scratch_types=[
            pltpu.VMEM((b_per_w, T, C), jnp.int32),
            pltpu.VMEM((NBUF, TH, C, N_EMBD), jnp.float32),
            pltpu.VMEM((POS_ROWS, N_EMBD), jnp.float32),
            pltpu.SemaphoreType.DMA,
            pltpu.SemaphoreType.DMA,
            pltpu.SemaphoreType.DMA,
            pltpu.SemaphoreType.DMA,
        ],
    )
    def gather_add(x_hbm, tok_hbm, pos_hbm, out_hbm, idx_v, rows_v, pos_v,
                   gsem0, gsem1, osem0, osem1):
        gsem = [gsem0, gsem1]
        osem = [osem0, osem1]
        cid = lax.axis_index("c")
        sid = lax.axis_index("s")
        wid = sid * 2 + cid
        b0 = wid * b_per_w
        pltpu.sync_copy(x_hbm.at[pl.ds(b0, b_per_w)], idx_v)
        pltpu.sync_copy(pos_hbm, pos_v)

        def fire_gathers(i, nbuf):
            bb = i // 2
            t0 = (i % 2) * TH
            for j in range(TH):
                pltpu.async_copy(
                    tok_hbm.at[idx_v.at[bb, t0 + j]],
                    rows_v.at[nbuf, j], gsem[nbuf])

        def drain_gathers(nbuf):
            pltpu.make_async_copy(
                out_hbm.at[0, pl.ds(0, TH)], rows_v.at[nbuf],
                gsem[nbuf]).wait()

        def drain_out(nbuf):
            pltpu.make_async_copy(
                rows_v.at[nbuf], out_hbm.at[0, pl.ds(0, TH)],
                osem[nbuf]).wait()

        def process(i, nbuf):
            drain_gathers(nbuf)

            def add_sub(s, carry):
                for cb in range(N_EMBD // LANES):
                    sl = pl.ds(cb * LANES, LANES)
                    pos_regs = [pos_v[r, sl] for r in range(POS_ROWS)]
                    for j in range(C):
                        rows_v[nbuf, s, j, sl] = (
                            rows_v[nbuf, s, j, sl] + pos_regs[j])
                return carry

            lax.fori_loop(0, TH, add_sub, 0)
            bb = i // 2
            t0 = (i % 2) * TH
            pltpu.async_copy(
                rows_v.at[nbuf], out_hbm.at[b0 + bb, pl.ds(t0, TH)],
                osem[nbuf])

        fire_gathers(0, 0)

        def outer(it, carry):
            i0 = it * NBUF
            process(i0, 0)

            @pl.when(it > 0)
            def _():
                drain_out(1)

            fire_gathers(i0 + 1, 1)
            process(i0 + 1, 1)

            @pl.when(it < nb // NBUF - 1)
            def _():
                drain_out(0)
                fire_gathers(i0 + NBUF, 0)

            return carry

        lax.fori_loop(0, nb // NBUF, outer, 0)
        drain_out(0)
        drain_out(1)

    return gather_add


def kernel(x, token_table, pos_table):
    B, T, C = x.shape
    return _make_gather(B, T, C)(x, token_table, pos_table)
